# baseline (speedup 1.0000x reference)
import functools

import jax
import jax.numpy as jnp
from jax import lax
from jax.experimental import pallas as pl
from jax.experimental.pallas import tpu as pltpu
from jax.experimental.pallas import tpu_sc as plsc

N_NODES = 20000
N_EDGES = 100000
H = 32
ED = 16
EH = 32
STEPS = 2

NC = 2
NS = 16
NW = NC * NS
IDX_BLK = 128
NBLK = 25
CHUNK = NBLK * IDX_BLK
EP = NW * CHUNK

ROWS_CORE = N_NODES // NC
AGG_ROWS = ROWS_CORE + 16
TRASH = ROWS_CORE
ZERO_SUB = AGG_ROWS // NS
OUT_SUB = ROWS_CORE // NS
ECH = EP // NS
SBLK = ECH // IDX_BLK
NPASS = 5
PBLK = SBLK // NPASS
PCH = PBLK * IDX_BLK

PK = 4
EPR = EP // PK
ER = N_EDGES // PK
TB = 256
TH = 512
TN = 1000

_f32 = jnp.float32


def _node_init_body(x_ref, w_ref, b_ref, o_ref, o16_ref):
    o = jnp.maximum(
        jnp.dot(x_ref[...], w_ref[...], preferred_element_type=_f32)
        + b_ref[...], 0.0)
    o_ref[...] = o
    o16_ref[...] = o.astype(jnp.bfloat16)


def _edge_hidden_body(ea_ref, w_ref, b_ref, o_ref):
    o_ref[...] = jnp.maximum(
        jnp.dot(ea_ref[...], w_ref[...], preferred_element_type=_f32)
        + b_ref[...], 0.0).astype(jnp.bfloat16)


def _msg_body(xj_ref, h_ref, w2_ref, b2_ref, s_ref, f_ref, o_ref):
    wet = jnp.dot(h_ref[...], w2_ref[...],
                  preferred_element_type=_f32).astype(jnp.bfloat16)
    wet = wet + b2_ref[...]
    xjrep = jnp.dot(xj_ref[...], s_ref[...],
                    preferred_element_type=_f32).astype(jnp.bfloat16)
    acc = xjrep * wet
    msg = jnp.dot(acc, f_ref[...], preferred_element_type=_f32)
    row = (pl.program_id(0) * TB
           + lax.broadcasted_iota(jnp.int32, (TB, 1), 0))
    o_ref[...] = jnp.where(row < ER, msg, 0.0)


def _update_body(p_ref, out_ref, root_ref, cb_ref, wm1_ref, wm2_ref,
                 bm_ref, extra_ref, o_ref, o16_ref):
    aggr = p_ref[...]
    out = out_ref[...]
    m = jnp.maximum(
        aggr + jnp.dot(out, root_ref[...], preferred_element_type=_f32)
        + cb_ref[...], 0.0)
    new = (jnp.dot(m, wm1_ref[...], preferred_element_type=_f32)
           + jnp.dot(out, wm2_ref[...], preferred_element_type=_f32)
           + bm_ref[...])
    res = new + extra_ref[...]
    o_ref[...] = res
    o16_ref[...] = res.astype(jnp.bfloat16)


_node_init = pl.pallas_call(
    _node_init_body,
    grid=(N_NODES // TN,),
    in_specs=[
        pl.BlockSpec((TN, H), lambda i: (i, 0)),
        pl.BlockSpec((H, H), lambda i: (0, 0)),
        pl.BlockSpec((1, H), lambda i: (0, 0)),
    ],
    out_specs=(pl.BlockSpec((TN, H), lambda i: (i, 0)),
               pl.BlockSpec((TN, H), lambda i: (i, 0))),
    out_shape=(jax.ShapeDtypeStruct((N_NODES, H), _f32),
               jax.ShapeDtypeStruct((N_NODES, H), jnp.bfloat16)),
)

_edge_hidden = pl.pallas_call(
    _edge_hidden_body,
    grid=(EPR // TH,),
    in_specs=[
        pl.BlockSpec((TH, PK * ED), lambda i: (i, 0)),
        pl.BlockSpec((PK * ED, PK * EH), lambda i: (0, 0)),
        pl.BlockSpec((1, PK * EH), lambda i: (0, 0)),
    ],
    out_specs=pl.BlockSpec((TH, PK * EH), lambda i: (i, 0)),
    out_shape=jax.ShapeDtypeStruct((EPR, PK * EH), jnp.bfloat16),
)

_msg = pl.pallas_call(
    _msg_body,
    grid=(EPR // TB,),
    in_specs=[
        pl.BlockSpec((TB, PK * H), lambda i: (i, 0)),
        pl.BlockSpec((TB, PK * EH), lambda i: (i, 0)),
        pl.BlockSpec((PK * EH, PK * H * H), lambda i: (0, 0)),
        pl.BlockSpec((1, PK * H * H), lambda i: (0, 0)),
        pl.BlockSpec((PK * H, PK * H * H), lambda i: (0, 0)),
        pl.BlockSpec((PK * H * H, PK * H), lambda i: (0, 0)),
    ],
    out_specs=pl.BlockSpec((TB, PK * H), lambda i: (i, 0)),
    out_shape=jax.ShapeDtypeStruct((EPR, PK * H), _f32),
)

_update = pl.pallas_call(
    _update_body,
    grid=(N_NODES // TN,),
    in_specs=[
        pl.BlockSpec((TN, H), lambda i: (i, 0)),
        pl.BlockSpec((TN, H), lambda i: (i, 0)),
        pl.BlockSpec((H, H), lambda i: (0, 0)),
        pl.BlockSpec((1, H), lambda i: (0, 0)),
        pl.BlockSpec((H, H), lambda i: (0, 0)),
        pl.BlockSpec((H, H), lambda i: (0, 0)),
        pl.BlockSpec((1, H), lambda i: (0, 0)),
        pl.BlockSpec((TN, H), lambda i: (i, 0)),
    ],
    out_specs=(pl.BlockSpec((TN, H), lambda i: (i, 0)),
               pl.BlockSpec((TN, H), lambda i: (i, 0))),
    out_shape=(jax.ShapeDtypeStruct((N_NODES, H), _f32),
               jax.ShapeDtypeStruct((N_NODES, H), jnp.bfloat16)),
)


def _gather_body(table_hbm, idx_hbm, out_hbm, idx_v, rows_v, sem):
    wid = lax.axis_index("s") * NC + lax.axis_index("c")
    pltpu.sync_copy(idx_hbm.at[wid], idx_v)
    for j in range(NBLK):
        pltpu.async_copy(table_hbm.at[idx_v.at[j]],
                         rows_v.at[pl.ds(j * IDX_BLK, IDX_BLK)], sem)
    for j in range(NBLK):
        pltpu.make_async_copy(table_hbm.at[idx_v.at[j]],
                              rows_v.at[pl.ds(j * IDX_BLK, IDX_BLK)],
                              sem).wait()
    pltpu.sync_copy(rows_v, out_hbm.at[pl.ds(wid * CHUNK, CHUNK)])


def _scatter_body(msg_hbm, idx_hbm, zeros_hbm, out_hbm,
                  idx_v, msg_v, msg_w, aggr_sh, sem, ssem):
    cid = lax.axis_index("c")
    sid = lax.axis_index("s")
    base = cid * ROWS_CORE
    pltpu.sync_copy(zeros_hbm, aggr_sh.at[pl.ds(sid * ZERO_SUB, ZERO_SUB)])
    pltpu.sync_copy(idx_hbm.at[sid], idx_v)

    def _remap(j, _):
        for l in range(IDX_BLK // 16):
            v = idx_v[j, pl.ds(l * 16, 16)]
            lv = v - base
            ok = (lv >= 0) & (lv < ROWS_CORE)
            idx_v[j, pl.ds(l * 16, 16)] = jnp.where(ok, lv, TRASH)
        return 0

    lax.fori_loop(0, SBLK, _remap, 0)
    plsc.subcore_barrier()

    def _stage(p, buf):
        pltpu.async_copy(
            msg_hbm.at[pl.ds(sid * ECH + p * PCH, PCH)], buf, ssem)

    def _fire(p, buf):
        for j in range(PBLK):
            pltpu.async_copy(buf.at[pl.ds(j * IDX_BLK, IDX_BLK)],
                             aggr_sh.at[idx_v.at[p * PBLK + j]],
                             sem, add=True)

    def _drain(p, buf):
        for j in range(PBLK):
            pltpu.make_async_copy(buf.at[pl.ds(j * IDX_BLK, IDX_BLK)],
                                  aggr_sh.at[idx_v.at[p * PBLK + j]],
                                  sem).wait()

    bufs = (msg_v, msg_w)
    _stage(0, bufs[0])
    for p in range(NPASS):
        buf = bufs[p % 2]
        pltpu.make_async_copy(
            msg_hbm.at[pl.ds(sid * ECH + p * PCH, PCH)], buf, ssem).wait()
        _fire(p, buf)
        if p + 1 < NPASS:
            _stage(p + 1, bufs[(p + 1) % 2])
        _drain(p, buf)
    plsc.subcore_barrier()
    pltpu.sync_copy(aggr_sh.at[pl.ds(sid * OUT_SUB, OUT_SUB)],
                    out_hbm.at[pl.ds(cid * ROWS_CORE + sid * OUT_SUB,
                                     OUT_SUB)])


@functools.lru_cache(maxsize=None)
def _sc_kernels():
    mesh = plsc.VectorSubcoreMesh(core_axis_name="c", subcore_axis_name="s",
                                  num_cores=NC, num_subcores=NS)
    params = pltpu.CompilerParams(use_tc_tiling_on_sc=False)
    gather = pl.kernel(
        _gather_body,
        out_type=jax.ShapeDtypeStruct((EP, H), jnp.bfloat16),
        mesh=mesh,
        compiler_params=params,
        scratch_types=[
            pltpu.VMEM((NBLK, IDX_BLK), jnp.int32),
            pltpu.VMEM((CHUNK, H), jnp.bfloat16),
            pltpu.SemaphoreType.DMA,
        ],
    )
    scatter = pl.kernel(
        _scatter_body,
        out_type=jax.ShapeDtypeStruct((N_NODES, H), _f32),
        mesh=mesh,
        compiler_params=params,
        scratch_types=[
            pltpu.VMEM((SBLK, IDX_BLK), jnp.int32),
            pltpu.VMEM((PCH, H), _f32),
            pltpu.VMEM((PCH, H), _f32),
            pltpu.VMEM_SHARED((AGG_ROWS, H), _f32),
            pltpu.SemaphoreType.DMA,
            pltpu.SemaphoreType.DMA,
        ],
    )
    return gather, scatter


def kernel(node_features, edge_attr, edge_index, W_in, b_in, W_e1, b_e1,
           W_e2, b_e2, root, conv_bias, W_msg, b_msg):
    pad = EP - N_EDGES
    src = jnp.concatenate(
        [edge_index[0], jnp.zeros((pad,), jnp.int32)]).reshape(
            NW, NBLK, IDX_BLK)
    dst = jnp.concatenate(
        [edge_index[1], jnp.zeros((pad,), jnp.int32)]).reshape(
            NS, SBLK, IDX_BLK)
    ea_p = jnp.concatenate(
        [edge_attr.reshape(ER, PK * ED),
         jnp.zeros((EPR - ER, PK * ED), _f32)], axis=0)
    zeros_sub = jnp.zeros((ZERO_SUB, H), _f32)
    no_extra = jnp.zeros((N_NODES, H), _f32)
    eye4 = jnp.eye(PK, dtype=_f32)
    sel = jnp.repeat(jnp.eye(H, dtype=_f32), H, axis=1)
    w1k = jnp.kron(eye4, W_e1)
    b1k = jnp.tile(b_e1, PK).reshape(1, PK * EH)
    w2k = jnp.kron(eye4, W_e2).astype(jnp.bfloat16)
    b2k = jnp.tile(b_e2, PK).reshape(1, PK * H * H).astype(jnp.bfloat16)
    selk = jnp.kron(eye4, sel).astype(jnp.bfloat16)
    foldk = jnp.kron(
        eye4, jnp.tile(jnp.eye(H, dtype=_f32), (H, 1))).astype(jnp.bfloat16)

    gather_sc, scatter_sc = _sc_kernels()
    out, out16 = _node_init(node_features, W_in, b_in.reshape(1, H))
    h = _edge_hidden(ea_p, w1k, b1k)

    for step in range(STEPS):
        xj = gather_sc(out16, src)
        msg = _msg(xj.reshape(EPR, PK * H), h, w2k, b2k, selk, foldk)
        aggr = scatter_sc(msg.reshape(EP, H), dst, zeros_sub)
        extra = node_features if step == STEPS - 1 else no_extra
        out, out16 = _update(aggr, out, root, conv_bias.reshape(1, H),
                             W_msg[:H], W_msg[H:], b_msg.reshape(1, H),
                             extra)
    return out


# speedup vs baseline: 1.0431x; 1.0431x over previous
import functools

import jax
import jax.numpy as jnp
from jax import lax
from jax.experimental import pallas as pl
from jax.experimental.pallas import tpu as pltpu
from jax.experimental.pallas import tpu_sc as plsc

N_NODES = 20000
N_EDGES = 100000
H = 32
ED = 16
EH = 32
STEPS = 2

NC = 2
NS = 16
NW = NC * NS
IDX_BLK = 128
NBLK = 25
CHUNK = NBLK * IDX_BLK
EP = NW * CHUNK

ROWS_CORE = N_NODES // NC
AGG_ROWS = ROWS_CORE + 16
TRASH = ROWS_CORE
ZERO_SUB = AGG_ROWS // NS
OUT_SUB = ROWS_CORE // NS
ECH = EP // NS
SBLK = ECH // IDX_BLK
NPASS = 5
PBLK = SBLK // NPASS
PCH = PBLK * IDX_BLK

PK = 4
EPR = EP // PK
ER = N_EDGES // PK
TB = 256
TH = 512
TN = 1000

_f32 = jnp.float32


def _node_init_body(x_ref, w_ref, b_ref, o_ref):
    o_ref[...] = jnp.maximum(
        jnp.dot(x_ref[...], w_ref[...], preferred_element_type=_f32)
        + b_ref[...], 0.0)


def _edge_hidden_body(ea_ref, w_ref, b_ref, o_ref):
    o_ref[...] = jnp.maximum(
        jnp.dot(ea_ref[...], w_ref[...], preferred_element_type=_f32)
        + b_ref[...], 0.0).astype(jnp.bfloat16)


def _msg_body(xj_ref, h_ref, w2_ref, b2_ref, s_ref, f_ref, o_ref):
    wet = jnp.dot(h_ref[...], w2_ref[...],
                  preferred_element_type=_f32).astype(jnp.bfloat16)
    wet = wet + b2_ref[...]
    xjrep = jnp.dot(xj_ref[...].astype(jnp.bfloat16), s_ref[...],
                    preferred_element_type=_f32).astype(jnp.bfloat16)
    acc = xjrep * wet
    msg = jnp.dot(acc, f_ref[...], preferred_element_type=_f32)
    row = (pl.program_id(0) * TB
           + lax.broadcasted_iota(jnp.int32, (TB, 1), 0))
    o_ref[...] = jnp.where(row < ER, msg, 0.0)


def _update_body(p_ref, out_ref, root_ref, cb_ref, wm1_ref, wm2_ref,
                 bm_ref, extra_ref, o_ref):
    aggr = p_ref[...]
    out = out_ref[...]
    m = jnp.maximum(
        aggr + jnp.dot(out, root_ref[...], preferred_element_type=_f32)
        + cb_ref[...], 0.0)
    new = (jnp.dot(m, wm1_ref[...], preferred_element_type=_f32)
           + jnp.dot(out, wm2_ref[...], preferred_element_type=_f32)
           + bm_ref[...])
    o_ref[...] = new + extra_ref[...]


_node_init = pl.pallas_call(
    _node_init_body,
    grid=(N_NODES // TN,),
    in_specs=[
        pl.BlockSpec((TN, H), lambda i: (i, 0)),
        pl.BlockSpec((H, H), lambda i: (0, 0)),
        pl.BlockSpec((1, H), lambda i: (0, 0)),
    ],
    out_specs=pl.BlockSpec((TN, H), lambda i: (i, 0)),
    out_shape=jax.ShapeDtypeStruct((N_NODES, H), _f32),
)

_edge_hidden = pl.pallas_call(
    _edge_hidden_body,
    grid=(EPR // TH,),
    in_specs=[
        pl.BlockSpec((TH, PK * ED), lambda i: (i, 0)),
        pl.BlockSpec((PK * ED, PK * EH), lambda i: (0, 0)),
        pl.BlockSpec((1, PK * EH), lambda i: (0, 0)),
    ],
    out_specs=pl.BlockSpec((TH, PK * EH), lambda i: (i, 0)),
    out_shape=jax.ShapeDtypeStruct((EPR, PK * EH), jnp.bfloat16),
)

_msg = pl.pallas_call(
    _msg_body,
    grid=(EPR // TB,),
    in_specs=[
        pl.BlockSpec((TB, PK * H), lambda i: (i, 0)),
        pl.BlockSpec((TB, PK * EH), lambda i: (i, 0)),
        pl.BlockSpec((PK * EH, PK * H * H), lambda i: (0, 0)),
        pl.BlockSpec((1, PK * H * H), lambda i: (0, 0)),
        pl.BlockSpec((PK * H, PK * H * H), lambda i: (0, 0)),
        pl.BlockSpec((PK * H * H, PK * H), lambda i: (0, 0)),
    ],
    out_specs=pl.BlockSpec((TB, PK * H), lambda i: (i, 0)),
    out_shape=jax.ShapeDtypeStruct((EPR, PK * H), _f32),
)

_update = pl.pallas_call(
    _update_body,
    grid=(N_NODES // TN,),
    in_specs=[
        pl.BlockSpec((TN, H), lambda i: (i, 0)),
        pl.BlockSpec((TN, H), lambda i: (i, 0)),
        pl.BlockSpec((H, H), lambda i: (0, 0)),
        pl.BlockSpec((1, H), lambda i: (0, 0)),
        pl.BlockSpec((H, H), lambda i: (0, 0)),
        pl.BlockSpec((H, H), lambda i: (0, 0)),
        pl.BlockSpec((1, H), lambda i: (0, 0)),
        pl.BlockSpec((TN, H), lambda i: (i, 0)),
    ],
    out_specs=pl.BlockSpec((TN, H), lambda i: (i, 0)),
    out_shape=jax.ShapeDtypeStruct((N_NODES, H), _f32),
)


def _gather_body(table_hbm, idx_hbm, out_hbm, idx_v, rows_v, sem):
    wid = lax.axis_index("s") * NC + lax.axis_index("c")
    pltpu.sync_copy(idx_hbm.at[wid], idx_v)
    for j in range(NBLK):
        pltpu.async_copy(table_hbm.at[idx_v.at[j]],
                         rows_v.at[pl.ds(j * IDX_BLK, IDX_BLK)], sem)
    for j in range(NBLK):
        pltpu.make_async_copy(table_hbm.at[idx_v.at[j]],
                              rows_v.at[pl.ds(j * IDX_BLK, IDX_BLK)],
                              sem).wait()
    pltpu.sync_copy(rows_v, out_hbm.at[pl.ds(wid * CHUNK, CHUNK)])


def _scatter_body(msg_hbm, idx_hbm, zeros_hbm, out_hbm,
                  idx_v, msg_v, msg_w, aggr_sh, sem, ssem):
    cid = lax.axis_index("c")
    sid = lax.axis_index("s")
    base = cid * ROWS_CORE
    pltpu.sync_copy(zeros_hbm, aggr_sh.at[pl.ds(sid * ZERO_SUB, ZERO_SUB)])
    pltpu.sync_copy(idx_hbm.at[sid], idx_v)

    def _remap(j, _):
        for l in range(IDX_BLK // 16):
            v = idx_v[j, pl.ds(l * 16, 16)]
            lv = v - base
            ok = (lv >= 0) & (lv < ROWS_CORE)
            idx_v[j, pl.ds(l * 16, 16)] = jnp.where(ok, lv, TRASH)
        return 0

    lax.fori_loop(0, SBLK, _remap, 0)
    plsc.subcore_barrier()

    def _stage(p, buf):
        pltpu.async_copy(
            msg_hbm.at[pl.ds(sid * ECH + p * PCH, PCH)], buf, ssem)

    def _fire(p, buf):
        for j in range(PBLK):
            pltpu.async_copy(buf.at[pl.ds(j * IDX_BLK, IDX_BLK)],
                             aggr_sh.at[idx_v.at[p * PBLK + j]],
                             sem, add=True)

    def _drain(p, buf):
        for j in range(PBLK):
            pltpu.make_async_copy(buf.at[pl.ds(j * IDX_BLK, IDX_BLK)],
                                  aggr_sh.at[idx_v.at[p * PBLK + j]],
                                  sem).wait()

    bufs = (msg_v, msg_w)
    _stage(0, bufs[0])
    for p in range(NPASS):
        buf = bufs[p % 2]
        pltpu.make_async_copy(
            msg_hbm.at[pl.ds(sid * ECH + p * PCH, PCH)], buf, ssem).wait()
        _fire(p, buf)
        if p + 1 < NPASS:
            _stage(p + 1, bufs[(p + 1) % 2])
        _drain(p, buf)
    plsc.subcore_barrier()
    pltpu.sync_copy(aggr_sh.at[pl.ds(sid * OUT_SUB, OUT_SUB)],
                    out_hbm.at[pl.ds(cid * ROWS_CORE + sid * OUT_SUB,
                                     OUT_SUB)])


@functools.lru_cache(maxsize=None)
def _sc_kernels():
    mesh = plsc.VectorSubcoreMesh(core_axis_name="c", subcore_axis_name="s",
                                  num_cores=NC, num_subcores=NS)
    params = pltpu.CompilerParams(use_tc_tiling_on_sc=False)
    gather = pl.kernel(
        _gather_body,
        out_type=jax.ShapeDtypeStruct((EP, H), _f32),
        mesh=mesh,
        compiler_params=params,
        scratch_types=[
            pltpu.VMEM((NBLK, IDX_BLK), jnp.int32),
            pltpu.VMEM((CHUNK, H), _f32),
            pltpu.SemaphoreType.DMA,
        ],
    )
    scatter = pl.kernel(
        _scatter_body,
        out_type=jax.ShapeDtypeStruct((N_NODES, H), _f32),
        mesh=mesh,
        compiler_params=params,
        scratch_types=[
            pltpu.VMEM((SBLK, IDX_BLK), jnp.int32),
            pltpu.VMEM((PCH, H), _f32),
            pltpu.VMEM((PCH, H), _f32),
            pltpu.VMEM_SHARED((AGG_ROWS, H), _f32),
            pltpu.SemaphoreType.DMA,
            pltpu.SemaphoreType.DMA,
        ],
    )
    return gather, scatter


def kernel(node_features, edge_attr, edge_index, W_in, b_in, W_e1, b_e1,
           W_e2, b_e2, root, conv_bias, W_msg, b_msg):
    pad = EP - N_EDGES
    src = jnp.concatenate(
        [edge_index[0], jnp.zeros((pad,), jnp.int32)]).reshape(
            NW, NBLK, IDX_BLK)
    dst = jnp.concatenate(
        [edge_index[1], jnp.zeros((pad,), jnp.int32)]).reshape(
            NS, SBLK, IDX_BLK)
    ea_p = jnp.concatenate(
        [edge_attr.reshape(ER, PK * ED),
         jnp.zeros((EPR - ER, PK * ED), _f32)], axis=0)
    zeros_sub = jnp.zeros((ZERO_SUB, H), _f32)
    no_extra = jnp.zeros((N_NODES, H), _f32)
    eye4 = jnp.eye(PK, dtype=_f32)
    sel = jnp.repeat(jnp.eye(H, dtype=_f32), H, axis=1)
    w1k = jnp.kron(eye4, W_e1)
    b1k = jnp.tile(b_e1, PK).reshape(1, PK * EH)
    w2k = jnp.kron(eye4, W_e2).astype(jnp.bfloat16)
    b2k = jnp.tile(b_e2, PK).reshape(1, PK * H * H).astype(jnp.bfloat16)
    selk = jnp.kron(eye4, sel).astype(jnp.bfloat16)
    foldk = jnp.kron(
        eye4, jnp.tile(jnp.eye(H, dtype=_f32), (H, 1))).astype(jnp.bfloat16)

    gather_sc, scatter_sc = _sc_kernels()
    out = _node_init(node_features, W_in, b_in.reshape(1, H))
    h = _edge_hidden(ea_p, w1k, b1k)

    for step in range(STEPS):
        xj = gather_sc(out, src)
        msg = _msg(xj.reshape(EPR, PK * H), h, w2k, b2k, selk, foldk)
        aggr = scatter_sc(msg.reshape(EP, H), dst, zeros_sub)
        extra = node_features if step == STEPS - 1 else no_extra
        out = _update(aggr, out, root, conv_bias.reshape(1, H),
                      W_msg[:H], W_msg[H:], b_msg.reshape(1, H), extra)
    return out
